# Initial kernel scaffold; baseline (speedup 1.0000x reference)
import jax
import jax.numpy as jnp
from jax.experimental import pallas as pl


def kernel(heter_edge_index, hyper_edge_index, emb, W1, b1, g1, be1, W2, b2, g2, be2, W4, b4, g4, be4):
    raise NotImplementedError("write your pallas kernel here")


# speedup vs baseline: 5.8059x; 5.8059x over previous
import functools

import jax
import jax.numpy as jnp
from jax import lax
from jax.experimental import pallas as pl
from jax.experimental.pallas import tpu as pltpu
from jax.experimental.pallas import tpu_sc as plsc

N = 10000
E = 320000
D = 128
NROWS = 10240
C = 128
NCHUNK = 2512
NSUB = 16
ROWS_PER_TILE = NROWS // NSUB
F32 = jnp.float32

_MESH = plsc.VectorSubcoreMesh(core_axis_name="c", subcore_axis_name="s")


def _make_sc_agg(dh, core_assign):
    njobs = len(core_assign)

    @functools.partial(
        pl.kernel,
        out_type=tuple(jax.ShapeDtypeStruct((NROWS, dh), F32) for _ in range(njobs)),
        mesh=_MESH,
        compiler_params=pltpu.CompilerParams(use_tc_tiling_on_sc=False),
        scratch_types=[
            pltpu.VMEM((C,), jnp.int32),
            pltpu.VMEM((C,), jnp.int32),
            pltpu.VMEM((C, dh), F32),
            pltpu.VMEM_SHARED((NROWS, dh), F32),
            pltpu.SemaphoreType.DMA,
        ],
    )
    def agg(*refs):
        ins = refs[: 4 * njobs]
        outs = refs[4 * njobs : 5 * njobs]
        gbuf, sbuf, rows, accum, sem = refs[5 * njobs :]
        cid = lax.axis_index("c")
        sid = lax.axis_index("s")
        row0 = sid * ROWS_PER_TILE

        def run(gidx_hbm, sidx_hbm, x_hbm, init_hbm, out_hbm):
            pltpu.sync_copy(
                init_hbm.at[pl.ds(row0, ROWS_PER_TILE)],
                accum.at[pl.ds(row0, ROWS_PER_TILE)],
            )
            plsc.subcore_barrier()

            @pl.loop(0, NCHUNK // NSUB)
            def _(i):
                ch = i * NSUB + sid
                pltpu.sync_copy(gidx_hbm.at[ch], gbuf)
                pltpu.sync_copy(sidx_hbm.at[ch], sbuf)
                pltpu.async_copy(x_hbm.at[gbuf], rows, sem).wait()
                pltpu.sync_copy(rows, accum.at[sbuf], add=True)

            plsc.subcore_barrier()
            pltpu.sync_copy(
                accum.at[pl.ds(row0, ROWS_PER_TILE)],
                out_hbm.at[pl.ds(row0, ROWS_PER_TILE)],
            )

        for j, cj in enumerate(core_assign):
            g, s, x, ini = ins[4 * j : 4 * j + 4]
            o = outs[j]

            @pl.when(cid == cj)
            def _(g=g, s=s, x=x, ini=ini, o=o):
                run(g, s, x, ini, o)

    return agg


_sc_agg2 = _make_sc_agg(D, (0, 1))
_sc_agg2h = _make_sc_agg(D // 2, (0, 1))
_sc_hist = _make_sc_agg(16, (0, 1, 1))


def _ln(x, g, b):
    mu = jnp.mean(x, axis=-1, keepdims=True)
    var = jnp.mean((x - mu) ** 2, axis=-1, keepdims=True)
    return (x - mu) * lax.rsqrt(var + 1e-5) * g + b


def _dot(a, b):
    return lax.dot_general(
        a, b, (((1,), (0,)), ((), ())),
        precision=lax.Precision.HIGHEST, preferred_element_type=F32,
    )


def _tc1_body(deg_ref, emb_ref, w1_ref, dinv_ref, ya_ref, yb_ref):
    dinv = lax.rsqrt(deg_ref[...] + 1.0)
    dinv_ref[...] = dinv
    y = _dot(emb_ref[...], w1_ref[...]) * dinv
    ya_ref[...] = y[:, :D]
    yb_ref[...] = y[:, D:]


def _tc2_body(dinv_ref, sa_ref, sb_ref, b1_ref, g1_ref, be1_ref, w2_ref,
              ya_ref, yb_ref):
    dinv = dinv_ref[...]
    agg = jnp.concatenate([sa_ref[...], sb_ref[...]], axis=1) * dinv + b1_ref[...]
    h = _ln(agg, g1_ref[...], be1_ref[...])
    y2 = _dot(h, w2_ref[...]) * dinv
    ya_ref[...] = y2[:, : D // 2]
    yb_ref[...] = y2[:, D // 2 :]


def _tc3_body(dinv_ref, sa_ref, sb_ref, b2_ref, g2_ref, be2_ref, w4_ref,
              ho_ref, xa_ref, xb_ref):
    dinv = dinv_ref[...]
    agg = jnp.concatenate([sa_ref[...], sb_ref[...]], axis=1) * dinv + b2_ref[...]
    ho = _ln(agg, g2_ref[...], be2_ref[...])
    ho_ref[...] = ho
    x4 = _dot(ho, w4_ref[...])
    xa_ref[...] = x4[:, : D // 2]
    xb_ref[...] = x4[:, D // 2 :]


def _tc4_body(be_ref, ha_ref, hb_ref, oa_ref, ob_ref):
    be = be_ref[...]
    binv = jnp.where(be > 0, 1.0 / be, 0.0)
    oa_ref[...] = ha_ref[...] * binv
    ob_ref[...] = hb_ref[...] * binv


def _tc5_body(dn_ref, ra_ref, rb_ref, b4_ref, g4_ref, be4_ref, out_ref):
    dn = dn_ref[...]
    dninv = jnp.where(dn > 0, 1.0 / dn, 0.0)
    agg = jnp.concatenate([ra_ref[...], rb_ref[...]], axis=1) * dninv + b4_ref[...]
    out_ref[...] = _ln(agg, g4_ref[...], be4_ref[...])


_RB = 2048


def _rspec(c_):
    return pl.BlockSpec((_RB, c_), lambda i: (i, 0))


def _fspec(r, c_):
    return pl.BlockSpec((r, c_), lambda i: (0, 0))


def _row_call(body, in_cols, out_cols):
    in_specs = [_rspec(c_) if isinstance(c_, int) else _fspec(*c_) for c_ in in_cols]
    outs = tuple(jax.ShapeDtypeStruct((NROWS, c_), F32) for c_ in out_cols)
    out_specs = tuple(_rspec(c_) for c_ in out_cols)
    return pl.pallas_call(
        body,
        grid=(NROWS // _RB,),
        in_specs=in_specs,
        out_specs=out_specs if len(out_cols) > 1 else out_specs[0],
        out_shape=outs if len(out_cols) > 1 else outs[0],
    )


_tc1 = _row_call(_tc1_body, [1, D, (D, 2 * D)], [1, D, D])
_tc2 = _row_call(_tc2_body, [1, D, D, (1, 2 * D), (1, 2 * D), (1, 2 * D), (2 * D, D)],
                 [D // 2, D // 2])
_tc3 = _row_call(_tc3_body, [1, D // 2, D // 2, (1, D), (1, D), (1, D), (D, D)],
                 [D, D // 2, D // 2])
_tc4 = _row_call(_tc4_body, [1, D // 2, D // 2], [D // 2, D // 2])
_tc5 = _row_call(_tc5_body, [1, D // 2, D // 2, (1, D), (1, D), (1, D)], [D])


def _pad_idx(a):
    pad = NCHUNK * C - E
    return jnp.concatenate(
        [a.astype(jnp.int32), jnp.full((pad,), N, jnp.int32)]
    ).reshape(NCHUNK, C)


def kernel(heter_edge_index, hyper_edge_index, emb, W1, b1, g1, be1,
           W2, b2, g2, be2, W4, b4, g4, be4):
    hs2 = _pad_idx(heter_edge_index[0])
    hd2 = _pad_idx(heter_edge_index[1])
    yn2 = _pad_idx(hyper_edge_index[0])
    yh2 = _pad_idx(hyper_edge_index[1])

    ones16 = jnp.ones((NROWS, 16), F32)
    z16 = jnp.zeros((NROWS, 16), F32)
    zh = jnp.zeros((NROWS, D // 2), F32)
    embp = jnp.pad(emb, ((0, NROWS - N), (0, 0)))
    b1r, g1r, be1r = b1[None, :], g1[None, :], be1[None, :]
    b2r, g2r, be2r = b2[None, :], g2[None, :], be2[None, :]
    b4r, g4r, be4r = b4[None, :], g4[None, :], be4[None, :]

    degh16, dn16, be16 = _sc_hist(
        hd2, hd2, ones16, z16,
        yn2, yn2, ones16, z16,
        yh2, yh2, ones16, z16,
    )
    degh = degh16[:, 0:1]
    dn = dn16[:, 0:1]
    behist = be16[:, 0:1]

    dinv, ya, yb = _tc1(degh, embp, W1)
    sa, sb = _sc_agg2(hs2, hd2, ya, ya, hs2, hd2, yb, yb)
    y2a, y2b = _tc2(dinv, sa, sb, b1r, g1r, be1r, W2)
    s2a, s2b = _sc_agg2h(hs2, hd2, y2a, y2a, hs2, hd2, y2b, y2b)
    hofull, x4a, x4b = _tc3(dinv, s2a, s2b, b2r, g2r, be2r, W4)
    ha, hb = _sc_agg2h(yn2, yh2, x4a, zh, yn2, yh2, x4b, zh)
    hsa, hsb = _tc4(behist, ha, hb)
    ra, rb = _sc_agg2h(yh2, yn2, hsa, zh, yh2, yn2, hsb, zh)
    hyfull = _tc5(dn, ra, rb, b4r, g4r, be4r)

    return hofull[:N], hyfull[:N]


# speedup vs baseline: 7.2925x; 1.2560x over previous
import functools

import jax
import jax.numpy as jnp
from jax import lax
from jax.experimental import pallas as pl
from jax.experimental.pallas import tpu as pltpu
from jax.experimental.pallas import tpu_sc as plsc

N = 10000
E = 320000
D = 128
NROWS = 10240
C = 128
NCHUNK = 2560
IB = 16
NSUB = 16
ROWS_PER_TILE = NROWS // NSUB
F32 = jnp.float32

_MESH = plsc.VectorSubcoreMesh(core_axis_name="c", subcore_axis_name="s")


def _make_sc_agg(dh, core_assign):
    njobs = len(core_assign)

    @functools.partial(
        pl.kernel,
        out_type=tuple(jax.ShapeDtypeStruct((NROWS, dh), F32) for _ in range(njobs)),
        mesh=_MESH,
        compiler_params=pltpu.CompilerParams(use_tc_tiling_on_sc=False),
        scratch_types=[
            pltpu.VMEM((2, IB, C), jnp.int32),
            pltpu.VMEM((2, IB, C), jnp.int32),
            pltpu.VMEM((C, dh), F32),
            pltpu.VMEM((C, dh), F32),
            pltpu.VMEM_SHARED((NROWS, dh), F32),
            pltpu.SemaphoreType.DMA,
            pltpu.SemaphoreType.DMA,
            pltpu.SemaphoreType.DMA,
            pltpu.SemaphoreType.DMA,
        ],
    )
    def agg(*refs):
        ins = refs[: 4 * njobs]
        outs = refs[4 * njobs : 5 * njobs]
        (gidx_l, sidx_l, buf_a, buf_b, accum,
         gsem_a, gsem_b, isem_a, isem_b) = refs[5 * njobs :]
        cid = lax.axis_index("c")
        sid = lax.axis_index("s")
        row0 = sid * ROWS_PER_TILE
        cpt = NCHUNK // NSUB
        nb = cpt // IB

        def run(gidx_hbm, sidx_hbm, x_hbm, init_hbm, out_hbm):
            pltpu.sync_copy(
                init_hbm.at[pl.ds(row0, ROWS_PER_TILE)],
                accum.at[pl.ds(row0, ROWS_PER_TILE)],
            )
            base = sid * cpt

            def start_idx(b, slot):
                gi, si, sem = (gidx_l.at[slot], sidx_l.at[slot],
                               isem_a if slot == 0 else isem_b)
                pltpu.async_copy(gidx_hbm.at[pl.ds(base + b * IB, IB)], gi, sem)
                pltpu.async_copy(sidx_hbm.at[pl.ds(base + b * IB, IB)], si, sem)

            def wait_idx(slot):
                gi, si, sem = (gidx_l.at[slot], sidx_l.at[slot],
                               isem_a if slot == 0 else isem_b)
                pltpu.make_async_copy(gidx_hbm.at[pl.ds(base, IB)], gi, sem).wait()
                pltpu.make_async_copy(sidx_hbm.at[pl.ds(base, IB)], si, sem).wait()

            def start_g(gi_v, i, buf, sem):
                pltpu.async_copy(x_hbm.at[gi_v.at[i]], buf, sem)

            def wait_g(gi_v, buf, sem):
                pltpu.make_async_copy(x_hbm.at[gi_v.at[0]], buf, sem).wait()

            def scat(si_v, i, buf):
                pltpu.sync_copy(buf, accum.at[si_v.at[i]], add=True)

            def process_batch(gi_v, si_v):
                start_g(gi_v, 0, buf_a, gsem_a)

                @pl.loop(0, IB // 2 - 1)
                def _(p):
                    i0 = 2 * p
                    start_g(gi_v, i0 + 1, buf_b, gsem_b)
                    wait_g(gi_v, buf_a, gsem_a)
                    scat(si_v, i0, buf_a)
                    start_g(gi_v, i0 + 2, buf_a, gsem_a)
                    wait_g(gi_v, buf_b, gsem_b)
                    scat(si_v, i0 + 1, buf_b)

                start_g(gi_v, IB - 1, buf_b, gsem_b)
                wait_g(gi_v, buf_a, gsem_a)
                scat(si_v, IB - 2, buf_a)
                wait_g(gi_v, buf_b, gsem_b)
                scat(si_v, IB - 1, buf_b)

            start_idx(0, 0)
            plsc.subcore_barrier()

            @pl.loop(0, nb // 2)
            def _(q):
                b0 = 2 * q
                start_idx(b0 + 1, 1)
                wait_idx(0)
                process_batch(gidx_l.at[0], sidx_l.at[0])

                @pl.when(q < nb // 2 - 1)
                def _():
                    start_idx(b0 + 2, 0)

                wait_idx(1)
                process_batch(gidx_l.at[1], sidx_l.at[1])

            plsc.subcore_barrier()
            pltpu.sync_copy(
                accum.at[pl.ds(row0, ROWS_PER_TILE)],
                out_hbm.at[pl.ds(row0, ROWS_PER_TILE)],
            )

        for j, cj in enumerate(core_assign):
            g, s, x, ini = ins[4 * j : 4 * j + 4]
            o = outs[j]

            @pl.when(cid == cj)
            def _(g=g, s=s, x=x, ini=ini, o=o):
                run(g, s, x, ini, o)

    return agg


_sc_agg2 = _make_sc_agg(D, (0, 1))
_sc_agg2h = _make_sc_agg(D // 2, (0, 1))
_sc_hist = _make_sc_agg(16, (0, 1, 1))


def _ln(x, g, b):
    mu = jnp.mean(x, axis=-1, keepdims=True)
    var = jnp.mean((x - mu) ** 2, axis=-1, keepdims=True)
    return (x - mu) * lax.rsqrt(var + 1e-5) * g + b


def _dot(a, b):
    return lax.dot_general(
        a, b, (((1,), (0,)), ((), ())),
        precision=lax.Precision.HIGHEST, preferred_element_type=F32,
    )


def _tc1_body(deg_ref, emb_ref, w1_ref, dinv_ref, ya_ref, yb_ref):
    dinv = lax.rsqrt(deg_ref[...] + 1.0)
    dinv_ref[...] = dinv
    y = _dot(emb_ref[...], w1_ref[...]) * dinv
    ya_ref[...] = y[:, :D]
    yb_ref[...] = y[:, D:]


def _tc2_body(dinv_ref, sa_ref, sb_ref, b1_ref, g1_ref, be1_ref, w2_ref,
              ya_ref, yb_ref):
    dinv = dinv_ref[...]
    agg = jnp.concatenate([sa_ref[...], sb_ref[...]], axis=1) * dinv + b1_ref[...]
    h = _ln(agg, g1_ref[...], be1_ref[...])
    y2 = _dot(h, w2_ref[...]) * dinv
    ya_ref[...] = y2[:, : D // 2]
    yb_ref[...] = y2[:, D // 2 :]


def _tc3_body(dinv_ref, sa_ref, sb_ref, b2_ref, g2_ref, be2_ref, w4_ref,
              ho_ref, xa_ref, xb_ref):
    dinv = dinv_ref[...]
    agg = jnp.concatenate([sa_ref[...], sb_ref[...]], axis=1) * dinv + b2_ref[...]
    ho = _ln(agg, g2_ref[...], be2_ref[...])
    ho_ref[...] = ho
    x4 = _dot(ho, w4_ref[...])
    xa_ref[...] = x4[:, : D // 2]
    xb_ref[...] = x4[:, D // 2 :]


def _tc4_body(be_ref, ha_ref, hb_ref, oa_ref, ob_ref):
    be = be_ref[...]
    binv = jnp.where(be > 0, 1.0 / be, 0.0)
    oa_ref[...] = ha_ref[...] * binv
    ob_ref[...] = hb_ref[...] * binv


def _tc5_body(dn_ref, ra_ref, rb_ref, b4_ref, g4_ref, be4_ref, out_ref):
    dn = dn_ref[...]
    dninv = jnp.where(dn > 0, 1.0 / dn, 0.0)
    agg = jnp.concatenate([ra_ref[...], rb_ref[...]], axis=1) * dninv + b4_ref[...]
    out_ref[...] = _ln(agg, g4_ref[...], be4_ref[...])


_RB = 2048


def _rspec(c_):
    return pl.BlockSpec((_RB, c_), lambda i: (i, 0))


def _fspec(r, c_):
    return pl.BlockSpec((r, c_), lambda i: (0, 0))


def _row_call(body, in_cols, out_cols):
    in_specs = [_rspec(c_) if isinstance(c_, int) else _fspec(*c_) for c_ in in_cols]
    outs = tuple(jax.ShapeDtypeStruct((NROWS, c_), F32) for c_ in out_cols)
    out_specs = tuple(_rspec(c_) for c_ in out_cols)
    return pl.pallas_call(
        body,
        grid=(NROWS // _RB,),
        in_specs=in_specs,
        out_specs=out_specs if len(out_cols) > 1 else out_specs[0],
        out_shape=outs if len(out_cols) > 1 else outs[0],
    )


_tc1 = _row_call(_tc1_body, [1, D, (D, 2 * D)], [1, D, D])
_tc2 = _row_call(_tc2_body, [1, D, D, (1, 2 * D), (1, 2 * D), (1, 2 * D), (2 * D, D)],
                 [D // 2, D // 2])
_tc3 = _row_call(_tc3_body, [1, D // 2, D // 2, (1, D), (1, D), (1, D), (D, D)],
                 [D, D // 2, D // 2])
_tc4 = _row_call(_tc4_body, [1, D // 2, D // 2], [D // 2, D // 2])
_tc5 = _row_call(_tc5_body, [1, D // 2, D // 2, (1, D), (1, D), (1, D)], [D])


def _pad_idx(a):
    pad = NCHUNK * C - E
    return jnp.concatenate(
        [a.astype(jnp.int32), jnp.full((pad,), N, jnp.int32)]
    ).reshape(NCHUNK, C)


def kernel(heter_edge_index, hyper_edge_index, emb, W1, b1, g1, be1,
           W2, b2, g2, be2, W4, b4, g4, be4):
    hs2 = _pad_idx(heter_edge_index[0])
    hd2 = _pad_idx(heter_edge_index[1])
    yn2 = _pad_idx(hyper_edge_index[0])
    yh2 = _pad_idx(hyper_edge_index[1])

    ones16 = jnp.ones((NROWS, 16), F32)
    z16 = jnp.zeros((NROWS, 16), F32)
    zh = jnp.zeros((NROWS, D // 2), F32)
    embp = jnp.pad(emb, ((0, NROWS - N), (0, 0)))
    b1r, g1r, be1r = b1[None, :], g1[None, :], be1[None, :]
    b2r, g2r, be2r = b2[None, :], g2[None, :], be2[None, :]
    b4r, g4r, be4r = b4[None, :], g4[None, :], be4[None, :]

    degh16, dn16, be16 = _sc_hist(
        hd2, hd2, ones16, z16,
        yn2, yn2, ones16, z16,
        yh2, yh2, ones16, z16,
    )
    degh = degh16[:, 0:1]
    dn = dn16[:, 0:1]
    behist = be16[:, 0:1]

    dinv, ya, yb = _tc1(degh, embp, W1)
    sa, sb = _sc_agg2(hs2, hd2, ya, ya, hs2, hd2, yb, yb)
    y2a, y2b = _tc2(dinv, sa, sb, b1r, g1r, be1r, W2)
    s2a, s2b = _sc_agg2h(hs2, hd2, y2a, y2a, hs2, hd2, y2b, y2b)
    hofull, x4a, x4b = _tc3(dinv, s2a, s2b, b2r, g2r, be2r, W4)
    ha, hb = _sc_agg2h(yn2, yh2, x4a, zh, yn2, yh2, x4b, zh)
    hsa, hsb = _tc4(behist, ha, hb)
    ra, rb = _sc_agg2h(yh2, yn2, hsa, zh, yh2, yn2, hsb, zh)
    hyfull = _tc5(dn, ra, rb, b4r, g4r, be4r)

    return hofull[:N], hyfull[:N]


# speedup vs baseline: 8.7300x; 1.1971x over previous
import functools

import jax
import jax.numpy as jnp
from jax import lax
from jax.experimental import pallas as pl
from jax.experimental.pallas import tpu as pltpu
from jax.experimental.pallas import tpu_sc as plsc

N = 10000
E = 320000
D = 128
NROWS = 10240
C = 128
NCHUNK = 2560
IB = 16
NSUB = 16
NBUF = 8
ROWS_PER_TILE = NROWS // NSUB
F32 = jnp.float32

_MESH = plsc.VectorSubcoreMesh(core_axis_name="c", subcore_axis_name="s")


def _make_sc_agg(dh, core_assign):
    njobs = len(core_assign)

    @functools.partial(
        pl.kernel,
        out_type=tuple(jax.ShapeDtypeStruct((NROWS, dh), F32) for _ in range(njobs)),
        mesh=_MESH,
        compiler_params=pltpu.CompilerParams(use_tc_tiling_on_sc=False),
        scratch_types=(
            [pltpu.VMEM((2, IB, C), jnp.int32),
             pltpu.VMEM((2, IB, C), jnp.int32)]
            + [pltpu.VMEM((C, dh), F32) for _ in range(NBUF)]
            + [pltpu.VMEM_SHARED((NROWS, dh), F32)]
            + [pltpu.SemaphoreType.DMA for _ in range(NBUF + 2)]
        ),
    )
    def agg(*refs):
        ins = refs[: 4 * njobs]
        outs = refs[4 * njobs : 5 * njobs]
        sc = refs[5 * njobs :]
        gidx_l, sidx_l = sc[0], sc[1]
        bufs = sc[2 : 2 + NBUF]
        accum = sc[2 + NBUF]
        sems = sc[3 + NBUF : 3 + 2 * NBUF]
        isem = sc[3 + 2 * NBUF : 5 + 2 * NBUF]
        cid = lax.axis_index("c")
        sid = lax.axis_index("s")
        row0 = sid * ROWS_PER_TILE
        cpt = NCHUNK // NSUB
        nb = cpt // IB

        def run(gidx_hbm, sidx_hbm, x_hbm, init_hbm, out_hbm):
            pltpu.sync_copy(
                init_hbm.at[pl.ds(row0, ROWS_PER_TILE)],
                accum.at[pl.ds(row0, ROWS_PER_TILE)],
            )
            base = sid * cpt

            def start_idx(b, slot):
                sem = isem[slot]
                pltpu.async_copy(
                    gidx_hbm.at[pl.ds(base + b * IB, IB)], gidx_l.at[slot], sem)
                pltpu.async_copy(
                    sidx_hbm.at[pl.ds(base + b * IB, IB)], sidx_l.at[slot], sem)

            def wait_idx(slot):
                sem = isem[slot]
                pltpu.make_async_copy(
                    gidx_hbm.at[pl.ds(base, IB)], gidx_l.at[slot], sem).wait()
                pltpu.make_async_copy(
                    sidx_hbm.at[pl.ds(base, IB)], sidx_l.at[slot], sem).wait()

            def start_g(gi_v, i, b):
                pltpu.async_copy(x_hbm.at[gi_v.at[i]], bufs[b], sems[b])

            def wait_g(gi_v, b):
                pltpu.make_async_copy(
                    x_hbm.at[gi_v.at[0]], bufs[b], sems[b]).wait()

            def start_s(si_v, i, b):
                pltpu.async_copy(
                    bufs[b], accum.at[si_v.at[i]], sems[b], add=True)

            def wait_s(si_v, b):
                pltpu.make_async_copy(
                    bufs[b], accum.at[si_v.at[0]], sems[b]).wait()

            def process_batch(gi_v, si_v):
                for k in range(4):
                    start_g(gi_v, k, k)
                for g in range(IB // 4):
                    s0 = (g % 2) * 4
                    n0 = ((g + 1) % 2) * 4
                    if g < IB // 4 - 1:
                        for k in range(4):
                            start_g(gi_v, 4 * (g + 1) + k, n0 + k)
                    for k in range(4):
                        wait_g(gi_v, s0 + k)
                        start_s(si_v, 4 * g + k, s0 + k)
                    for k in range(4):
                        wait_s(si_v, s0 + k)

            start_idx(0, 0)
            plsc.subcore_barrier()

            @pl.loop(0, nb // 2)
            def _(q):
                b0 = 2 * q
                start_idx(b0 + 1, 1)
                wait_idx(0)
                process_batch(gidx_l.at[0], sidx_l.at[0])

                @pl.when(q < nb // 2 - 1)
                def _():
                    start_idx(b0 + 2, 0)

                wait_idx(1)
                process_batch(gidx_l.at[1], sidx_l.at[1])

            plsc.subcore_barrier()
            pltpu.sync_copy(
                accum.at[pl.ds(row0, ROWS_PER_TILE)],
                out_hbm.at[pl.ds(row0, ROWS_PER_TILE)],
            )

        for j, cj in enumerate(core_assign):
            g, s, x, ini = ins[4 * j : 4 * j + 4]
            o = outs[j]

            @pl.when(cid == cj)
            def _(g=g, s=s, x=x, ini=ini, o=o):
                run(g, s, x, ini, o)

    return agg


_sc_agg = _make_sc_agg(D // 2, (0, 1))
_sc_hist = _make_sc_agg(16, (0, 1, 1))


def _ln(x, g, b):
    mu = jnp.mean(x, axis=-1, keepdims=True)
    var = jnp.mean((x - mu) ** 2, axis=-1, keepdims=True)
    return (x - mu) * lax.rsqrt(var + 1e-5) * g + b


def _dot(a, b):
    return lax.dot_general(
        a, b, (((1,), (0,)), ((), ())),
        precision=lax.Precision.HIGHEST, preferred_element_type=F32,
    )


def _tc1_body(deg_ref, emb_ref, dinv_ref, ya_ref, yb_ref):
    dinv = lax.rsqrt(deg_ref[...] + 1.0)
    dinv_ref[...] = dinv
    y = emb_ref[...] * dinv
    ya_ref[...] = y[:, : D // 2]
    yb_ref[...] = y[:, D // 2 :]


def _tc2_body(dinv_ref, sa_ref, sb_ref, b1_ref, g1_ref, be1_ref, w1_ref,
              w2_ref, ya_ref, yb_ref):
    dinv = dinv_ref[...]
    agg = jnp.concatenate([sa_ref[...], sb_ref[...]], axis=1) * dinv
    x1 = _dot(agg, w1_ref[...]) + b1_ref[...]
    h = _ln(x1, g1_ref[...], be1_ref[...])
    y2 = _dot(h, w2_ref[...]) * dinv
    ya_ref[...] = y2[:, : D // 2]
    yb_ref[...] = y2[:, D // 2 :]


def _tc3_body(dinv_ref, sa_ref, sb_ref, b2_ref, g2_ref, be2_ref, w4_ref,
              ho_ref, xa_ref, xb_ref):
    dinv = dinv_ref[...]
    agg = jnp.concatenate([sa_ref[...], sb_ref[...]], axis=1) * dinv + b2_ref[...]
    ho = _ln(agg, g2_ref[...], be2_ref[...])
    ho_ref[...] = ho
    x4 = _dot(ho, w4_ref[...])
    xa_ref[...] = x4[:, : D // 2]
    xb_ref[...] = x4[:, D // 2 :]


def _tc4_body(be_ref, ha_ref, hb_ref, oa_ref, ob_ref):
    be = be_ref[...]
    binv = jnp.where(be > 0, 1.0 / be, 0.0)
    oa_ref[...] = ha_ref[...] * binv
    ob_ref[...] = hb_ref[...] * binv


def _tc5_body(dn_ref, ra_ref, rb_ref, b4_ref, g4_ref, be4_ref, out_ref):
    dn = dn_ref[...]
    dninv = jnp.where(dn > 0, 1.0 / dn, 0.0)
    agg = jnp.concatenate([ra_ref[...], rb_ref[...]], axis=1) * dninv + b4_ref[...]
    out_ref[...] = _ln(agg, g4_ref[...], be4_ref[...])


_RB = 2048


def _rspec(c_):
    return pl.BlockSpec((_RB, c_), lambda i: (i, 0))


def _fspec(r, c_):
    return pl.BlockSpec((r, c_), lambda i: (0, 0))


def _row_call(body, in_cols, out_cols):
    in_specs = [_rspec(c_) if isinstance(c_, int) else _fspec(*c_) for c_ in in_cols]
    outs = tuple(jax.ShapeDtypeStruct((NROWS, c_), F32) for c_ in out_cols)
    out_specs = tuple(_rspec(c_) for c_ in out_cols)
    return pl.pallas_call(
        body,
        grid=(NROWS // _RB,),
        in_specs=in_specs,
        out_specs=out_specs if len(out_cols) > 1 else out_specs[0],
        out_shape=outs if len(out_cols) > 1 else outs[0],
    )


_tc1 = _row_call(_tc1_body, [1, D], [1, D // 2, D // 2])
_tc2 = _row_call(
    _tc2_body,
    [1, D // 2, D // 2, (1, 2 * D), (1, 2 * D), (1, 2 * D), (D, 2 * D), (2 * D, D)],
    [D // 2, D // 2])
_tc3 = _row_call(_tc3_body, [1, D // 2, D // 2, (1, D), (1, D), (1, D), (D, D)],
                 [D, D // 2, D // 2])
_tc4 = _row_call(_tc4_body, [1, D // 2, D // 2], [D // 2, D // 2])
_tc5 = _row_call(_tc5_body, [1, D // 2, D // 2, (1, D), (1, D), (1, D)], [D])


def _pad_idx(a):
    pad = NCHUNK * C - E
    return jnp.concatenate(
        [a.astype(jnp.int32), jnp.full((pad,), N, jnp.int32)]
    ).reshape(NCHUNK, C)


def kernel(heter_edge_index, hyper_edge_index, emb, W1, b1, g1, be1,
           W2, b2, g2, be2, W4, b4, g4, be4):
    hs2 = _pad_idx(heter_edge_index[0])
    hd2 = _pad_idx(heter_edge_index[1])
    yn2 = _pad_idx(hyper_edge_index[0])
    yh2 = _pad_idx(hyper_edge_index[1])

    ones16 = jnp.ones((NROWS, 16), F32)
    z16 = jnp.zeros((NROWS, 16), F32)
    zh = jnp.zeros((NROWS, D // 2), F32)
    embp = jnp.pad(emb, ((0, NROWS - N), (0, 0)))
    b1r, g1r, be1r = b1[None, :], g1[None, :], be1[None, :]
    b2r, g2r, be2r = b2[None, :], g2[None, :], be2[None, :]
    b4r, g4r, be4r = b4[None, :], g4[None, :], be4[None, :]

    degh16, dn16, be16 = _sc_hist(
        hd2, hd2, ones16, z16,
        yn2, yn2, ones16, z16,
        yh2, yh2, ones16, z16,
    )
    degh = degh16[:, 0:1]
    dn = dn16[:, 0:1]
    behist = be16[:, 0:1]

    dinv, ya, yb = _tc1(degh, embp)
    sa, sb = _sc_agg(hs2, hd2, ya, ya, hs2, hd2, yb, yb)
    y2a, y2b = _tc2(dinv, sa, sb, b1r, g1r, be1r, W1, W2)
    s2a, s2b = _sc_agg(hs2, hd2, y2a, y2a, hs2, hd2, y2b, y2b)
    hofull, x4a, x4b = _tc3(dinv, s2a, s2b, b2r, g2r, be2r, W4)
    ha, hb = _sc_agg(yn2, yh2, x4a, zh, yn2, yh2, x4b, zh)
    hsa, hsb = _tc4(behist, ha, hb)
    ra, rb = _sc_agg(yh2, yn2, hsa, zh, yh2, yn2, hsb, zh)
    hyfull = _tc5(dn, ra, rb, b4r, g4r, be4r)

    return hofull[:N], hyfull[:N]


# speedup vs baseline: 9.3819x; 1.0747x over previous
import functools

import jax
import jax.numpy as jnp
from jax import lax
from jax.experimental import pallas as pl
from jax.experimental.pallas import tpu as pltpu
from jax.experimental.pallas import tpu_sc as plsc

N = 10000
E = 320000
D = 128
NROWS = 10240
C = 128
NCHUNK = 2560
IB = 16
NSUB = 16
NBUF = 8
ROWS_PER_TILE = NROWS // NSUB
F32 = jnp.float32

_MESH = plsc.VectorSubcoreMesh(core_axis_name="c", subcore_axis_name="s")


def _make_sc_agg(dh, jobs, ib=IB):
    njobs = len(jobs)

    @functools.partial(
        pl.kernel,
        out_type=tuple(jax.ShapeDtypeStruct((NROWS, dh), F32) for _ in range(njobs)),
        mesh=_MESH,
        compiler_params=pltpu.CompilerParams(use_tc_tiling_on_sc=False),
        scratch_types=(
            [pltpu.VMEM((2, ib, C), jnp.int32),
             pltpu.VMEM((2, ib, C), jnp.int32)]
            + [pltpu.VMEM((C, dh), F32) for _ in range(NBUF)]
            + [pltpu.VMEM_SHARED((NROWS, dh), F32)]
            + [pltpu.SemaphoreType.DMA for _ in range(NBUF + 2)]
        ),
    )
    def agg(*refs):
        ins = refs[: 4 * njobs]
        outs = refs[4 * njobs : 5 * njobs]
        sc = refs[5 * njobs :]
        gidx_l, sidx_l = sc[0], sc[1]
        bufs = sc[2 : 2 + NBUF]
        accum = sc[2 + NBUF]
        sems = sc[3 + NBUF : 3 + 2 * NBUF]
        isem = sc[3 + 2 * NBUF : 5 + 2 * NBUF]
        cid = lax.axis_index("c")
        sid = lax.axis_index("s")
        row0 = sid * ROWS_PER_TILE

        def run(gidx_hbm, sidx_hbm, x_hbm, init_hbm, out_hbm, lo, hi, gather):
            pltpu.sync_copy(
                init_hbm.at[pl.ds(row0, ROWS_PER_TILE)],
                accum.at[pl.ds(row0, ROWS_PER_TILE)],
            )
            cpt = (hi - lo) // NSUB
            nb = cpt // ib
            base = lo + sid * cpt
            if not gather:
                pltpu.sync_copy(x_hbm.at[pl.ds(0, C)], bufs[0])

            def start_idx(b, slot):
                sem = isem[slot]
                if gather:
                    pltpu.async_copy(
                        gidx_hbm.at[pl.ds(base + b * ib, ib)], gidx_l.at[slot], sem)
                pltpu.async_copy(
                    sidx_hbm.at[pl.ds(base + b * ib, ib)], sidx_l.at[slot], sem)

            def wait_idx(slot):
                sem = isem[slot]
                if gather:
                    pltpu.make_async_copy(
                        gidx_hbm.at[pl.ds(base, ib)], gidx_l.at[slot], sem).wait()
                pltpu.make_async_copy(
                    sidx_hbm.at[pl.ds(base, ib)], sidx_l.at[slot], sem).wait()

            def start_g(gi_v, i, b):
                pltpu.async_copy(x_hbm.at[gi_v.at[i]], bufs[b], sems[b])

            def wait_g(gi_v, b):
                pltpu.make_async_copy(
                    x_hbm.at[gi_v.at[0]], bufs[b], sems[b]).wait()

            def start_s(si_v, i, b, src):
                pltpu.async_copy(
                    bufs[src], accum.at[si_v.at[i]], sems[b], add=True)

            def wait_s(si_v, b, src):
                pltpu.make_async_copy(
                    bufs[src], accum.at[si_v.at[0]], sems[b]).wait()

            def process_batch(gi_v, si_v):
                for k in range(4):
                    start_g(gi_v, k, k)
                for g in range(ib // 4):
                    s0 = (g % 2) * 4
                    n0 = ((g + 1) % 2) * 4
                    if g < ib // 4 - 1:
                        for k in range(4):
                            start_g(gi_v, 4 * (g + 1) + k, n0 + k)
                    for k in range(4):
                        wait_g(gi_v, s0 + k)
                        start_s(si_v, 4 * g + k, s0 + k, s0 + k)
                    for k in range(4):
                        wait_s(si_v, s0 + k, s0 + k)

            def process_batch_const(si_v):
                for i in range(ib):
                    if i >= 4:
                        wait_s(si_v, i % 4, 0)
                    start_s(si_v, i, i % 4, 0)
                for k in range(max(0, ib - 4), ib):
                    wait_s(si_v, k % 4, 0)

            start_idx(0, 0)
            plsc.subcore_barrier()

            @pl.loop(0, nb // 2)
            def _(q):
                b0 = 2 * q
                start_idx(b0 + 1, 1)
                wait_idx(0)
                if gather:
                    process_batch(gidx_l.at[0], sidx_l.at[0])
                else:
                    process_batch_const(sidx_l.at[0])

                @pl.when(q < nb // 2 - 1)
                def _():
                    start_idx(b0 + 2, 0)

                wait_idx(1)
                if gather:
                    process_batch(gidx_l.at[1], sidx_l.at[1])
                else:
                    process_batch_const(sidx_l.at[1])

            plsc.subcore_barrier()
            pltpu.sync_copy(
                accum.at[pl.ds(row0, ROWS_PER_TILE)],
                out_hbm.at[pl.ds(row0, ROWS_PER_TILE)],
            )

        for j, (cj, lo, hi, gather) in enumerate(jobs):
            g, s, x, ini = ins[4 * j : 4 * j + 4]
            o = outs[j]

            @pl.when(cid == cj)
            def _(g=g, s=s, x=x, ini=ini, o=o, lo=lo, hi=hi, gather=gather):
                run(g, s, x, ini, o, lo, hi, gather)

    return agg


_sc_agg = _make_sc_agg(D // 2, [(0, 0, NCHUNK, True), (1, 0, NCHUNK, True)])
_sc_hist = _make_sc_agg(
    8,
    [(0, 0, NCHUNK, False), (1, 0, NCHUNK, False),
     (0, 0, NCHUNK // 2, False), (1, NCHUNK // 2, NCHUNK, False)],
    ib=8)


def _ln(x, g, b):
    mu = jnp.mean(x, axis=-1, keepdims=True)
    var = jnp.mean((x - mu) ** 2, axis=-1, keepdims=True)
    return (x - mu) * lax.rsqrt(var + 1e-5) * g + b


def _dot(a, b):
    return lax.dot_general(
        a, b, (((1,), (0,)), ((), ())),
        precision=lax.Precision.HIGHEST, preferred_element_type=F32,
    )


def _tc1_body(deg_ref, emb_ref, dinv_ref, ya_ref, yb_ref):
    dinv = lax.rsqrt(deg_ref[...] + 1.0)
    dinv_ref[...] = dinv
    y = emb_ref[...] * dinv
    ya_ref[...] = y[:, : D // 2]
    yb_ref[...] = y[:, D // 2 :]


def _tc2_body(dinv_ref, sa_ref, sb_ref, b1_ref, g1_ref, be1_ref, w1_ref,
              w2_ref, ya_ref, yb_ref):
    dinv = dinv_ref[...]
    agg = jnp.concatenate([sa_ref[...], sb_ref[...]], axis=1) * dinv
    x1 = _dot(agg, w1_ref[...]) + b1_ref[...]
    h = _ln(x1, g1_ref[...], be1_ref[...])
    y2 = _dot(h, w2_ref[...]) * dinv
    ya_ref[...] = y2[:, : D // 2]
    yb_ref[...] = y2[:, D // 2 :]


def _tc3_body(dinv_ref, sa_ref, sb_ref, b2_ref, g2_ref, be2_ref, w4_ref,
              ho_ref, xa_ref, xb_ref):
    dinv = dinv_ref[...]
    agg = jnp.concatenate([sa_ref[...], sb_ref[...]], axis=1) * dinv + b2_ref[...]
    ho = _ln(agg, g2_ref[...], be2_ref[...])
    ho_ref[...] = ho
    x4 = _dot(ho, w4_ref[...])
    xa_ref[...] = x4[:, : D // 2]
    xb_ref[...] = x4[:, D // 2 :]


def _tc4_body(be0_ref, be1_ref, ha_ref, hb_ref, oa_ref, ob_ref):
    be = be0_ref[...] + be1_ref[...]
    binv = jnp.where(be > 0, 1.0 / be, 0.0)
    oa_ref[...] = ha_ref[...] * binv
    ob_ref[...] = hb_ref[...] * binv


def _tc5_body(dn_ref, ra_ref, rb_ref, b4_ref, g4_ref, be4_ref, out_ref):
    dn = dn_ref[...]
    dninv = jnp.where(dn > 0, 1.0 / dn, 0.0)
    agg = jnp.concatenate([ra_ref[...], rb_ref[...]], axis=1) * dninv + b4_ref[...]
    out_ref[...] = _ln(agg, g4_ref[...], be4_ref[...])


_RB = 2048


def _rspec(c_):
    return pl.BlockSpec((_RB, c_), lambda i: (i, 0))


def _fspec(r, c_):
    return pl.BlockSpec((r, c_), lambda i: (0, 0))


def _row_call(body, in_cols, out_cols):
    in_specs = [_rspec(c_) if isinstance(c_, int) else _fspec(*c_) for c_ in in_cols]
    outs = tuple(jax.ShapeDtypeStruct((NROWS, c_), F32) for c_ in out_cols)
    out_specs = tuple(_rspec(c_) for c_ in out_cols)
    return pl.pallas_call(
        body,
        grid=(NROWS // _RB,),
        in_specs=in_specs,
        out_specs=out_specs if len(out_cols) > 1 else out_specs[0],
        out_shape=outs if len(out_cols) > 1 else outs[0],
    )


_tc1 = _row_call(_tc1_body, [1, D], [1, D // 2, D // 2])
_tc2 = _row_call(
    _tc2_body,
    [1, D // 2, D // 2, (1, 2 * D), (1, 2 * D), (1, 2 * D), (D, 2 * D), (2 * D, D)],
    [D // 2, D // 2])
_tc3 = _row_call(_tc3_body, [1, D // 2, D // 2, (1, D), (1, D), (1, D), (D, D)],
                 [D, D // 2, D // 2])
_tc4 = _row_call(_tc4_body, [1, 1, D // 2, D // 2], [D // 2, D // 2])
_tc5 = _row_call(_tc5_body, [1, D // 2, D // 2, (1, D), (1, D), (1, D)], [D])


def _pad_idx(a):
    pad = NCHUNK * C - E
    return jnp.concatenate(
        [a.astype(jnp.int32), jnp.full((pad,), N, jnp.int32)]
    ).reshape(NCHUNK, C)


def kernel(heter_edge_index, hyper_edge_index, emb, W1, b1, g1, be1,
           W2, b2, g2, be2, W4, b4, g4, be4):
    hs2 = _pad_idx(heter_edge_index[0])
    hd2 = _pad_idx(heter_edge_index[1])
    yn2 = _pad_idx(hyper_edge_index[0])
    yh2 = _pad_idx(hyper_edge_index[1])

    ones8 = jnp.ones((C, 8), F32)
    z8 = jnp.zeros((NROWS, 8), F32)
    zh = jnp.zeros((NROWS, D // 2), F32)
    embp = jnp.pad(emb, ((0, NROWS - N), (0, 0)))
    b1r, g1r, be1r = b1[None, :], g1[None, :], be1[None, :]
    b2r, g2r, be2r = b2[None, :], g2[None, :], be2[None, :]
    b4r, g4r, be4r = b4[None, :], g4[None, :], be4[None, :]

    degh8, dn8, beh0, beh1 = _sc_hist(
        hd2, hd2, ones8, z8,
        yn2, yn2, ones8, z8,
        yh2, yh2, ones8, z8,
        yh2, yh2, ones8, z8,
    )
    degh = degh8[:, 0:1]
    dn = dn8[:, 0:1]
    be0 = beh0[:, 0:1]
    be1 = beh1[:, 0:1]

    dinv, ya, yb = _tc1(degh, embp)
    sa, sb = _sc_agg(hs2, hd2, ya, ya, hs2, hd2, yb, yb)
    y2a, y2b = _tc2(dinv, sa, sb, b1r, g1r, be1r, W1, W2)
    s2a, s2b = _sc_agg(hs2, hd2, y2a, y2a, hs2, hd2, y2b, y2b)
    hofull, x4a, x4b = _tc3(dinv, s2a, s2b, b2r, g2r, be2r, W4)
    ha, hb = _sc_agg(yn2, yh2, x4a, zh, yn2, yh2, x4b, zh)
    hsa, hsb = _tc4(be0, be1, ha, hb)
    ra, rb = _sc_agg(yh2, yn2, hsa, zh, yh2, yn2, hsb, zh)
    hyfull = _tc5(dn, ra, rb, b4r, g4r, be4r)

    return hofull[:N], hyfull[:N]


# speedup vs baseline: 9.5432x; 1.0172x over previous
import functools

import jax
import jax.numpy as jnp
from jax import lax
from jax.experimental import pallas as pl
from jax.experimental.pallas import tpu as pltpu
from jax.experimental.pallas import tpu_sc as plsc

N = 10000
E = 320000
D = 128
NROWS = 10240
C = 128
NCHUNK = 2560
IB = 16
NSUB = 16
NBUF = 8
ROWS_PER_TILE = NROWS // NSUB
F32 = jnp.float32

_MESH = plsc.VectorSubcoreMesh(core_axis_name="c", subcore_axis_name="s")


def _make_sc_agg(dh, jobs, ib=IB):
    njobs = len(jobs)

    @functools.partial(
        pl.kernel,
        out_type=tuple(jax.ShapeDtypeStruct((NROWS, dh), F32) for _ in range(njobs)),
        mesh=_MESH,
        compiler_params=pltpu.CompilerParams(use_tc_tiling_on_sc=False),
        scratch_types=(
            [pltpu.VMEM((2, ib, C), jnp.int32),
             pltpu.VMEM((2, ib, C), jnp.int32)]
            + [pltpu.VMEM((C, dh), F32) for _ in range(NBUF)]
            + [pltpu.VMEM_SHARED((NROWS, dh), F32)]
            + [pltpu.SemaphoreType.DMA for _ in range(NBUF + 2)]
        ),
    )
    def agg(*refs):
        ins = refs[: 4 * njobs]
        outs = refs[4 * njobs : 5 * njobs]
        sc = refs[5 * njobs :]
        gidx_l, sidx_l = sc[0], sc[1]
        bufs = sc[2 : 2 + NBUF]
        accum = sc[2 + NBUF]
        sems = sc[3 + NBUF : 3 + 2 * NBUF]
        isem = sc[3 + 2 * NBUF : 5 + 2 * NBUF]
        cid = lax.axis_index("c")
        sid = lax.axis_index("s")
        row0 = sid * ROWS_PER_TILE

        def run(gidx_hbm, sidx_hbm, x_hbm, init_hbm, out_hbm, lo, hi, gather):
            pltpu.sync_copy(
                init_hbm.at[pl.ds(row0, ROWS_PER_TILE)],
                accum.at[pl.ds(row0, ROWS_PER_TILE)],
            )
            cpt = (hi - lo) // NSUB
            nb = cpt // ib
            base = lo + sid * cpt
            if not gather:
                pltpu.sync_copy(x_hbm.at[pl.ds(0, C)], bufs[0])

            def start_idx(b, slot):
                sem = isem[slot]
                if gather:
                    pltpu.async_copy(
                        gidx_hbm.at[pl.ds(base + b * ib, ib)], gidx_l.at[slot], sem)
                pltpu.async_copy(
                    sidx_hbm.at[pl.ds(base + b * ib, ib)], sidx_l.at[slot], sem)

            def wait_idx(slot):
                sem = isem[slot]
                if gather:
                    pltpu.make_async_copy(
                        gidx_hbm.at[pl.ds(base, ib)], gidx_l.at[slot], sem).wait()
                pltpu.make_async_copy(
                    sidx_hbm.at[pl.ds(base, ib)], sidx_l.at[slot], sem).wait()

            def start_g(gi_v, i, b):
                pltpu.async_copy(x_hbm.at[gi_v.at[i]], bufs[b], sems[b])

            def wait_g(gi_v, b):
                pltpu.make_async_copy(
                    x_hbm.at[gi_v.at[0]], bufs[b], sems[b]).wait()

            def start_s(si_v, i, b, src):
                pltpu.async_copy(
                    bufs[src], accum.at[si_v.at[i]], sems[b], add=True)

            def wait_s(si_v, b, src):
                pltpu.make_async_copy(
                    bufs[src], accum.at[si_v.at[0]], sems[b]).wait()

            def process_batch(gi_v, si_v):
                for k in range(4):
                    start_g(gi_v, k, k)
                for g in range(ib // 4):
                    s0 = (g % 2) * 4
                    n0 = ((g + 1) % 2) * 4
                    if g < ib // 4 - 1:
                        for k in range(4):
                            start_g(gi_v, 4 * (g + 1) + k, n0 + k)
                    for k in range(4):
                        wait_g(gi_v, s0 + k)
                        start_s(si_v, 4 * g + k, s0 + k, s0 + k)
                    for k in range(4):
                        wait_s(si_v, s0 + k, s0 + k)

            def process_batch_const(si_v):
                for i in range(ib):
                    if i >= 4:
                        wait_s(si_v, i % 4, 0)
                    start_s(si_v, i, i % 4, 0)
                for k in range(max(0, ib - 4), ib):
                    wait_s(si_v, k % 4, 0)

            start_idx(0, 0)
            plsc.subcore_barrier()

            @pl.loop(0, nb // 2)
            def _(q):
                b0 = 2 * q
                start_idx(b0 + 1, 1)
                wait_idx(0)
                if gather:
                    process_batch(gidx_l.at[0], sidx_l.at[0])
                else:
                    process_batch_const(sidx_l.at[0])

                @pl.when(q < nb // 2 - 1)
                def _():
                    start_idx(b0 + 2, 0)

                wait_idx(1)
                if gather:
                    process_batch(gidx_l.at[1], sidx_l.at[1])
                else:
                    process_batch_const(sidx_l.at[1])

            plsc.subcore_barrier()
            pltpu.sync_copy(
                accum.at[pl.ds(row0, ROWS_PER_TILE)],
                out_hbm.at[pl.ds(row0, ROWS_PER_TILE)],
            )

        for j, (cj, lo, hi, gather) in enumerate(jobs):
            g, s, x, ini = ins[4 * j : 4 * j + 4]
            o = outs[j]

            @pl.when(cid == cj)
            def _(g=g, s=s, x=x, ini=ini, o=o, lo=lo, hi=hi, gather=gather):
                run(g, s, x, ini, o, lo, hi, gather)

    return agg


_sc_agg = _make_sc_agg(D // 2, [(0, 0, NCHUNK, True), (1, 0, NCHUNK, True)])
_sc_hist = _make_sc_agg(
    8,
    [(0, 0, NCHUNK, False), (1, 0, NCHUNK, False),
     (0, 0, NCHUNK // 2, False), (1, NCHUNK // 2, NCHUNK, False)],
    ib=8)


def _ln(x, g, b):
    mu = jnp.mean(x, axis=-1, keepdims=True)
    var = jnp.mean((x - mu) ** 2, axis=-1, keepdims=True)
    return (x - mu) * lax.rsqrt(var + 1e-5) * g + b


def _dot(a, b):
    return lax.dot_general(
        a, b, (((1,), (0,)), ((), ())),
        preferred_element_type=F32,
    )


def _tc1_body(deg_ref, emb_ref, dinv_ref, ya_ref, yb_ref):
    dinv = lax.rsqrt(deg_ref[...] + 1.0)
    dinv_ref[...] = dinv
    y = emb_ref[...] * dinv
    ya_ref[...] = y[:, : D // 2]
    yb_ref[...] = y[:, D // 2 :]


def _tc2_body(dinv_ref, sa_ref, sb_ref, b1_ref, g1_ref, be1_ref, w1_ref,
              w2_ref, ya_ref, yb_ref):
    dinv = dinv_ref[...]
    agg = jnp.concatenate([sa_ref[...], sb_ref[...]], axis=1) * dinv
    x1 = _dot(agg, w1_ref[...]) + b1_ref[...]
    h = _ln(x1, g1_ref[...], be1_ref[...])
    y2 = _dot(h, w2_ref[...]) * dinv
    ya_ref[...] = y2[:, : D // 2]
    yb_ref[...] = y2[:, D // 2 :]


def _tc3_body(dinv_ref, sa_ref, sb_ref, b2_ref, g2_ref, be2_ref, w4_ref,
              ho_ref, xa_ref, xb_ref):
    dinv = dinv_ref[...]
    agg = jnp.concatenate([sa_ref[...], sb_ref[...]], axis=1) * dinv + b2_ref[...]
    ho = _ln(agg, g2_ref[...], be2_ref[...])
    ho_ref[...] = ho
    x4 = _dot(ho, w4_ref[...])
    xa_ref[...] = x4[:, : D // 2]
    xb_ref[...] = x4[:, D // 2 :]


def _tc4_body(be0_ref, be1_ref, ha_ref, hb_ref, oa_ref, ob_ref):
    be = be0_ref[...] + be1_ref[...]
    binv = jnp.where(be > 0, 1.0 / be, 0.0)
    oa_ref[...] = ha_ref[...] * binv
    ob_ref[...] = hb_ref[...] * binv


def _tc5_body(dn_ref, ra_ref, rb_ref, b4_ref, g4_ref, be4_ref, out_ref):
    dn = dn_ref[...]
    dninv = jnp.where(dn > 0, 1.0 / dn, 0.0)
    agg = jnp.concatenate([ra_ref[...], rb_ref[...]], axis=1) * dninv + b4_ref[...]
    out_ref[...] = _ln(agg, g4_ref[...], be4_ref[...])


_RB = 2048


def _rspec(c_):
    return pl.BlockSpec((_RB, c_), lambda i: (i, 0))


def _fspec(r, c_):
    return pl.BlockSpec((r, c_), lambda i: (0, 0))


def _row_call(body, in_cols, out_cols):
    in_specs = [_rspec(c_) if isinstance(c_, int) else _fspec(*c_) for c_ in in_cols]
    outs = tuple(jax.ShapeDtypeStruct((NROWS, c_), F32) for c_ in out_cols)
    out_specs = tuple(_rspec(c_) for c_ in out_cols)
    return pl.pallas_call(
        body,
        grid=(NROWS // _RB,),
        in_specs=in_specs,
        out_specs=out_specs if len(out_cols) > 1 else out_specs[0],
        out_shape=outs if len(out_cols) > 1 else outs[0],
    )


_tc1 = _row_call(_tc1_body, [1, D], [1, D // 2, D // 2])
_tc2 = _row_call(
    _tc2_body,
    [1, D // 2, D // 2, (1, 2 * D), (1, 2 * D), (1, 2 * D), (D, 2 * D), (2 * D, D)],
    [D // 2, D // 2])
_tc3 = _row_call(_tc3_body, [1, D // 2, D // 2, (1, D), (1, D), (1, D), (D, D)],
                 [D, D // 2, D // 2])
_tc4 = _row_call(_tc4_body, [1, 1, D // 2, D // 2], [D // 2, D // 2])
_tc5 = _row_call(_tc5_body, [1, D // 2, D // 2, (1, D), (1, D), (1, D)], [D])


def _pad_idx(a):
    pad = NCHUNK * C - E
    return jnp.concatenate(
        [a.astype(jnp.int32), jnp.full((pad,), N, jnp.int32)]
    ).reshape(NCHUNK, C)


def kernel(heter_edge_index, hyper_edge_index, emb, W1, b1, g1, be1,
           W2, b2, g2, be2, W4, b4, g4, be4):
    hs2 = _pad_idx(heter_edge_index[0])
    hd2 = _pad_idx(heter_edge_index[1])
    yn2 = _pad_idx(hyper_edge_index[0])
    yh2 = _pad_idx(hyper_edge_index[1])

    ones8 = jnp.ones((C, 8), F32)
    z8 = jnp.zeros((NROWS, 8), F32)
    zh = jnp.zeros((NROWS, D // 2), F32)
    embp = jnp.pad(emb, ((0, NROWS - N), (0, 0)))
    b1r, g1r, be1r = b1[None, :], g1[None, :], be1[None, :]
    b2r, g2r, be2r = b2[None, :], g2[None, :], be2[None, :]
    b4r, g4r, be4r = b4[None, :], g4[None, :], be4[None, :]

    degh8, dn8, beh0, beh1 = _sc_hist(
        hd2, hd2, ones8, z8,
        yn2, yn2, ones8, z8,
        yh2, yh2, ones8, z8,
        yh2, yh2, ones8, z8,
    )
    degh = degh8[:, 0:1]
    dn = dn8[:, 0:1]
    be0 = beh0[:, 0:1]
    be1 = beh1[:, 0:1]

    dinv, ya, yb = _tc1(degh, embp)
    sa, sb = _sc_agg(hs2, hd2, ya, ya, hs2, hd2, yb, yb)
    y2a, y2b = _tc2(dinv, sa, sb, b1r, g1r, be1r, W1, W2)
    s2a, s2b = _sc_agg(hs2, hd2, y2a, y2a, hs2, hd2, y2b, y2b)
    hofull, x4a, x4b = _tc3(dinv, s2a, s2b, b2r, g2r, be2r, W4)
    ha, hb = _sc_agg(yn2, yh2, x4a, zh, yn2, yh2, x4b, zh)
    hsa, hsb = _tc4(be0, be1, ha, hb)
    ra, rb = _sc_agg(yh2, yn2, hsa, zh, yh2, yn2, hsb, zh)
    hyfull = _tc5(dn, ra, rb, b4r, g4r, be4r)

    return hofull[:N], hyfull[:N]
